# baseline (speedup 1.0000x reference)
import functools

import jax
import jax.numpy as jnp
from jax import lax
from jax.experimental import pallas as pl
from jax.experimental.pallas import tpu as pltpu
from jax.experimental.pallas import tpu_sc as plsc

N = 10000
D_IN = 128
D_G = 128
D_H = 256
D_L = 64
E_TOT = 320000

_NC = 2
_NS = 16
_NW = _NC * _NS
_DH2 = D_G // _NC
_CH = 80
_RPT = E_TOT // _NS // _CH
_NSEC = 5
_SEC = _RPT // _NSEC
_EPS = 1e-5


def _mm_bias_body(x_ref, w_ref, b_ref, o_ref):
    u = (
        jnp.dot(x_ref[...], w_ref[...], preferred_element_type=jnp.float32)
        + b_ref[...]
    )
    o_ref[0] = u[:, :_DH2]
    o_ref[1] = u[:, _DH2:]


def _mm_bias_split(x, w, b):
    n, k = x.shape
    m = w.shape[1]
    nb = 10
    blk = n // nb
    return pl.pallas_call(
        _mm_bias_body,
        out_shape=jax.ShapeDtypeStruct((_NC, n, _DH2), jnp.float32),
        grid=(nb,),
        in_specs=[
            pl.BlockSpec((blk, k), lambda i: (i, 0)),
            pl.BlockSpec((k, m), lambda i: (0, 0)),
            pl.BlockSpec((1, m), lambda i: (0, 0)),
        ],
        out_specs=pl.BlockSpec((_NC, blk, _DH2), lambda i: (0, i, 0)),
    )(x, w, b.reshape(1, m))


def _sc_agg(hs, src4, dst4, w4, zeros_nd):
    mesh = plsc.VectorSubcoreMesh(core_axis_name="c", subcore_axis_name="s")

    @functools.partial(
        pl.kernel,
        out_type=jax.ShapeDtypeStruct((_NC, N, _DH2), jnp.float32),
        mesh=mesh,
        scratch_types=[
            pltpu.VMEM((_SEC, _CH), jnp.int32),
            pltpu.VMEM((_SEC, _CH), jnp.int32),
            pltpu.VMEM((_SEC, _CH), jnp.float32),
            pltpu.VMEM((_CH, _DH2), jnp.float32),
            pltpu.VMEM((_CH, _DH2), jnp.float32),
            pltpu.VMEM_SHARED((N, _DH2), jnp.float32),
            pltpu.VMEM_SHARED((N, _DH2), jnp.float32),
            pltpu.SemaphoreType.DMA,
            pltpu.SemaphoreType.DMA,
            pltpu.SemaphoreType.DMA,
            pltpu.SemaphoreType.DMA,
        ],
        compiler_params=pltpu.CompilerParams(use_tc_tiling_on_sc=False),
    )
    def k(hs_hbm, src_hbm, dst_hbm, w_hbm, z_hbm, out_hbm,
          src_v, dst_v, w_v, buf0, buf1, acc, hsh,
          gsem0, gsem1, ssem0, ssem1):
        c = lax.axis_index("c")
        s = lax.axis_index("s")

        zb = s * 624
        pltpu.sync_copy(z_hbm.at[pl.ds(zb, 624)], acc.at[pl.ds(zb, 624)])
        pltpu.sync_copy(hs_hbm.at[c].at[pl.ds(zb, 624)],
                        hsh.at[pl.ds(zb, 624)])

        @pl.when(s == _NS - 1)
        def _():
            pltpu.sync_copy(z_hbm.at[pl.ds(9984, 16)],
                            acc.at[pl.ds(9984, 16)])
            pltpu.sync_copy(hs_hbm.at[c].at[pl.ds(9984, 16)],
                            hsh.at[pl.ds(9984, 16)])

        plsc.subcore_barrier()

        def start_gather(g, buf, sem):
            pltpu.async_copy(hsh.at[src_v.at[g]], buf, sem)

        def wait_gather(buf, sem):
            pltpu.make_async_copy(hsh.at[src_v.at[0]], buf, sem).wait()

        def multiply(g, buf):
            def egroup(g16, _):
                w16 = w_v[g, pl.ds(g16 * 16, 16)]
                for l in range(16):
                    wv = jnp.full((16,), w16[l], dtype=jnp.float32)
                    r = g16 * 16 + l
                    for jj in range(_DH2 // 16):
                        sl = pl.ds(jj * 16, 16)
                        buf[r, sl] = buf[r, sl] * wv
                return 0
            lax.fori_loop(0, _CH // 16, egroup, 0)

        def start_scatter(g, buf, sem):
            pltpu.async_copy(buf, acc.at[dst_v.at[g]], sem, add=True)

        def wait_scatter(buf, sem):
            pltpu.make_async_copy(buf, acc.at[dst_v.at[0]], sem).wait()

        def section(sec, _):
            pltpu.sync_copy(src_hbm.at[s, sec], src_v)
            pltpu.sync_copy(dst_hbm.at[s, sec], dst_v)
            pltpu.sync_copy(w_hbm.at[s, sec], w_v)

            start_gather(0, buf0, gsem0)

            def pair(k2, _):
                g0 = 2 * k2

                @pl.when(k2 > 0)
                def _():
                    wait_scatter(buf1, ssem1)

                start_gather(g0 + 1, buf1, gsem1)
                wait_gather(buf0, gsem0)
                multiply(g0, buf0)
                start_scatter(g0, buf0, ssem0)

                wait_scatter(buf0, ssem0)

                @pl.when(g0 + 2 < _SEC)
                def _():
                    start_gather(g0 + 2, buf0, gsem0)

                wait_gather(buf1, gsem1)
                multiply(g0 + 1, buf1)
                start_scatter(g0 + 1, buf1, ssem1)
                return 0

            lax.fori_loop(0, _SEC // 2, pair, 0)
            wait_scatter(buf1, ssem1)
            return 0

        lax.fori_loop(0, _NSEC, section, 0)

        plsc.subcore_barrier()
        pltpu.sync_copy(acc.at[pl.ds(zb, 624)],
                        out_hbm.at[c].at[pl.ds(zb, 624)])

        @pl.when(s == _NS - 1)
        def _():
            pltpu.sync_copy(acc.at[pl.ds(9984, 16)],
                            out_hbm.at[c].at[pl.ds(9984, 16)])

    return k(hs, src4, dst4, w4, zeros_nd)


def _stats_body(p_ref, agg_ref, st_ref, acc_ref):
    i = pl.program_id(0)
    a = jnp.concatenate([p_ref[0], p_ref[1]], axis=1)
    agg_ref[...] = a
    st = jnp.concatenate(
        [jnp.sum(a, axis=0, keepdims=True),
         jnp.sum(a * a, axis=0, keepdims=True)], axis=0)

    @pl.when(i == 0)
    def _():
        acc_ref[...] = st

    @pl.when(i != 0)
    def _():
        acc_ref[...] = acc_ref[...] + st

    @pl.when(i == pl.num_programs(0) - 1)
    def _():
        st_ref[...] = acc_ref[...]


def _stats(parts):
    nb = 10
    blk = N // nb
    return pl.pallas_call(
        _stats_body,
        out_shape=[
            jax.ShapeDtypeStruct((N, D_G), jnp.float32),
            jax.ShapeDtypeStruct((2, D_G), jnp.float32),
        ],
        grid=(nb,),
        in_specs=[pl.BlockSpec((_NC, blk, _DH2), lambda i: (0, i, 0))],
        out_specs=[
            pl.BlockSpec((blk, D_G), lambda i: (i, 0)),
            pl.BlockSpec((2, D_G), lambda i: (0, 0)),
        ],
        scratch_shapes=[pltpu.VMEM((2, D_G), jnp.float32)],
    )(parts)


def _mid_body(agg_ref, st_ref, g0_ref, b0_ref, w1_ref, b1_ref,
              u_ref, st1_ref, acc_ref):
    i = pl.program_id(0)
    st = st_ref[...]
    mean = st[0:1] * (1.0 / N)
    var = st[1:2] * (1.0 / N) - mean * mean
    inv = lax.rsqrt(var + _EPS)
    h0 = jax.nn.softplus(
        (agg_ref[...] - mean) * inv * g0_ref[...] + b0_ref[...])
    u = jnp.dot(h0, w1_ref[...], preferred_element_type=jnp.float32) + b1_ref[...]
    u_ref[...] = u
    st1 = jnp.concatenate(
        [jnp.sum(u, axis=0, keepdims=True),
         jnp.sum(u * u, axis=0, keepdims=True)], axis=0)

    @pl.when(i == 0)
    def _():
        acc_ref[...] = st1

    @pl.when(i != 0)
    def _():
        acc_ref[...] = acc_ref[...] + st1

    @pl.when(i == pl.num_programs(0) - 1)
    def _():
        st1_ref[...] = acc_ref[...]


def _mid(agg, st0, gamma0, beta0, W1, b1):
    nb = 10
    blk = N // nb
    return pl.pallas_call(
        _mid_body,
        out_shape=[
            jax.ShapeDtypeStruct((N, D_H), jnp.float32),
            jax.ShapeDtypeStruct((2, D_H), jnp.float32),
        ],
        grid=(nb,),
        in_specs=[
            pl.BlockSpec((blk, D_G), lambda i: (i, 0)),
            pl.BlockSpec((2, D_G), lambda i: (0, 0)),
            pl.BlockSpec((1, D_G), lambda i: (0, 0)),
            pl.BlockSpec((1, D_G), lambda i: (0, 0)),
            pl.BlockSpec((D_G, D_H), lambda i: (0, 0)),
            pl.BlockSpec((1, D_H), lambda i: (0, 0)),
        ],
        out_specs=[
            pl.BlockSpec((blk, D_H), lambda i: (i, 0)),
            pl.BlockSpec((2, D_H), lambda i: (0, 0)),
        ],
        scratch_shapes=[pltpu.VMEM((2, D_H), jnp.float32)],
    )(agg, st0, gamma0.reshape(1, -1), beta0.reshape(1, -1), W1,
      b1.reshape(1, -1))


def _final_body(u_ref, st_ref, g1_ref, b1_ref, w21_ref, b21_ref,
                w22_ref, b22_ref, mu_ref, lv_ref):
    st = st_ref[...]
    mean = st[0:1] * (1.0 / N)
    var = st[1:2] * (1.0 / N) - mean * mean
    inv = lax.rsqrt(var + _EPS)
    h1 = jax.nn.softplus(
        (u_ref[...] - mean) * inv * g1_ref[...] + b1_ref[...])
    mu_ref[...] = (
        jnp.dot(h1, w21_ref[...], preferred_element_type=jnp.float32)
        + b21_ref[...])
    lv_ref[...] = (
        jnp.dot(h1, w22_ref[...], preferred_element_type=jnp.float32)
        + b22_ref[...])


def _final(u, st1, gamma1, beta1, W21, b21, W22, b22):
    nb = 10
    blk = N // nb
    return pl.pallas_call(
        _final_body,
        out_shape=[
            jax.ShapeDtypeStruct((N, D_L), jnp.float32),
            jax.ShapeDtypeStruct((N, D_L), jnp.float32),
        ],
        grid=(nb,),
        in_specs=[
            pl.BlockSpec((blk, D_H), lambda i: (i, 0)),
            pl.BlockSpec((2, D_H), lambda i: (0, 0)),
            pl.BlockSpec((1, D_H), lambda i: (0, 0)),
            pl.BlockSpec((1, D_H), lambda i: (0, 0)),
            pl.BlockSpec((D_H, D_L), lambda i: (0, 0)),
            pl.BlockSpec((1, D_L), lambda i: (0, 0)),
            pl.BlockSpec((D_H, D_L), lambda i: (0, 0)),
            pl.BlockSpec((1, D_L), lambda i: (0, 0)),
        ],
        out_specs=[
            pl.BlockSpec((blk, D_L), lambda i: (i, 0)),
            pl.BlockSpec((blk, D_L), lambda i: (i, 0)),
        ],
    )(u, st1, gamma1.reshape(1, -1), beta1.reshape(1, -1),
      W21, b21.reshape(1, -1), W22, b22.reshape(1, -1))


def kernel(x, edge_index, edge_weight, Wg, bg, gamma0, beta0, W1, b1,
           gamma1, beta1, W21, b21, W22, b22):
    hs = _mm_bias_split(x, Wg, bg)
    dst4 = edge_index[0].reshape(_NS, _NSEC, _SEC, _CH)
    src4 = edge_index[1].reshape(_NS, _NSEC, _SEC, _CH)
    w4 = edge_weight.reshape(_NS, _NSEC, _SEC, _CH)
    zeros_nd = jnp.zeros((N, _DH2), jnp.float32)
    parts = _sc_agg(hs, src4, dst4, w4, zeros_nd)
    agg, st0 = _stats(parts)
    u, st1 = _mid(agg, st0, gamma0, beta0, W1, b1)
    mu, logvar = _final(u, st1, gamma1, beta1, W21, b21, W22, b22)
    return (mu, logvar)


# speedup vs baseline: 1.0241x; 1.0241x over previous
import functools

import jax
import jax.numpy as jnp
from jax import lax
from jax.experimental import pallas as pl
from jax.experimental.pallas import tpu as pltpu
from jax.experimental.pallas import tpu_sc as plsc

N = 10000
D_IN = 128
D_G = 128
D_H = 256
D_L = 64
E_TOT = 320000

_NC = 2
_NS = 16
_NW = _NC * _NS
_DH2 = D_G // _NC
_CH = 80
_RPT = E_TOT // _NS // _CH
_NSEC = 5
_SEC = _RPT // _NSEC
_EPS = 1e-5


def _mm_bias_body(x_ref, w_ref, b_ref, o_ref):
    u = (
        jnp.dot(x_ref[...], w_ref[...], preferred_element_type=jnp.float32)
        + b_ref[...]
    )
    o_ref[0] = u[:, :_DH2]
    o_ref[1] = u[:, _DH2:]


def _mm_bias_split(x, w, b):
    n, k = x.shape
    m = w.shape[1]
    nb = 10
    blk = n // nb
    return pl.pallas_call(
        _mm_bias_body,
        out_shape=jax.ShapeDtypeStruct((_NC, n, _DH2), jnp.float32),
        grid=(nb,),
        in_specs=[
            pl.BlockSpec((blk, k), lambda i: (i, 0)),
            pl.BlockSpec((k, m), lambda i: (0, 0)),
            pl.BlockSpec((1, m), lambda i: (0, 0)),
        ],
        out_specs=pl.BlockSpec((_NC, blk, _DH2), lambda i: (0, i, 0)),
    )(x, w, b.reshape(1, m))


def _sc_agg(hs, src4, dst4, w4, zeros_nd):
    mesh = plsc.VectorSubcoreMesh(core_axis_name="c", subcore_axis_name="s")

    @functools.partial(
        pl.kernel,
        out_type=jax.ShapeDtypeStruct((_NC, N, _DH2), jnp.float32),
        mesh=mesh,
        scratch_types=[
            pltpu.VMEM((_SEC, _CH), jnp.int32),
            pltpu.VMEM((_SEC, _CH), jnp.int32),
            pltpu.VMEM((_SEC, _CH), jnp.float32),
            pltpu.VMEM((_CH, _DH2), jnp.float32),
            pltpu.VMEM((_CH, _DH2), jnp.float32),
            pltpu.VMEM_SHARED((N, _DH2), jnp.float32),
            pltpu.SemaphoreType.DMA,
            pltpu.SemaphoreType.DMA,
            pltpu.SemaphoreType.DMA,
            pltpu.SemaphoreType.DMA,
        ],
        compiler_params=pltpu.CompilerParams(use_tc_tiling_on_sc=False),
    )
    def k(hs_hbm, src_hbm, dst_hbm, w_hbm, z_hbm, out_hbm,
          src_v, dst_v, w_v, buf0, buf1, acc,
          gsem0, gsem1, ssem0, ssem1):
        c = lax.axis_index("c")
        s = lax.axis_index("s")

        zb = s * 624
        pltpu.sync_copy(z_hbm.at[pl.ds(zb, 624)], acc.at[pl.ds(zb, 624)])

        @pl.when(s == _NS - 1)
        def _():
            pltpu.sync_copy(z_hbm.at[pl.ds(9984, 16)],
                            acc.at[pl.ds(9984, 16)])

        plsc.subcore_barrier()

        hhalf = hs_hbm.at[c]

        def start_gather(g, buf, sem):
            pltpu.async_copy(hhalf.at[src_v.at[g]], buf, sem)

        def wait_gather(buf, sem):
            pltpu.make_async_copy(hhalf.at[src_v.at[0]], buf, sem).wait()

        def multiply(g, buf):
            def egroup(g16, _):
                w16 = w_v[g, pl.ds(g16 * 16, 16)]
                for l in range(16):
                    wv = jnp.full((16,), w16[l], dtype=jnp.float32)
                    r = g16 * 16 + l
                    for jj in range(_DH2 // 16):
                        sl = pl.ds(jj * 16, 16)
                        buf[r, sl] = buf[r, sl] * wv
                return 0
            lax.fori_loop(0, _CH // 16, egroup, 0)

        def start_scatter(g, buf, sem):
            pltpu.async_copy(buf, acc.at[dst_v.at[g]], sem, add=True)

        def wait_scatter(buf, sem):
            pltpu.make_async_copy(buf, acc.at[dst_v.at[0]], sem).wait()

        def section(sec, _):
            pltpu.sync_copy(src_hbm.at[s, sec], src_v)
            pltpu.sync_copy(dst_hbm.at[s, sec], dst_v)
            pltpu.sync_copy(w_hbm.at[s, sec], w_v)

            start_gather(0, buf0, gsem0)

            def pair(k2, _):
                g0 = 2 * k2

                @pl.when(k2 > 0)
                def _():
                    wait_scatter(buf1, ssem1)

                start_gather(g0 + 1, buf1, gsem1)
                wait_gather(buf0, gsem0)
                multiply(g0, buf0)
                start_scatter(g0, buf0, ssem0)

                wait_scatter(buf0, ssem0)

                @pl.when(g0 + 2 < _SEC)
                def _():
                    start_gather(g0 + 2, buf0, gsem0)

                wait_gather(buf1, gsem1)
                multiply(g0 + 1, buf1)
                start_scatter(g0 + 1, buf1, ssem1)
                return 0

            lax.fori_loop(0, _SEC // 2, pair, 0)
            wait_scatter(buf1, ssem1)
            return 0

        lax.fori_loop(0, _NSEC, section, 0)

        plsc.subcore_barrier()
        pltpu.sync_copy(acc.at[pl.ds(zb, 624)],
                        out_hbm.at[c].at[pl.ds(zb, 624)])

        @pl.when(s == _NS - 1)
        def _():
            pltpu.sync_copy(acc.at[pl.ds(9984, 16)],
                            out_hbm.at[c].at[pl.ds(9984, 16)])

    return k(hs, src4, dst4, w4, zeros_nd)


_HNB = 10
_HBLK = N // _HNB


def _head_body(p_ref, g0_ref, b0_ref, w1_ref, b1_ref, g1_ref, b1n_ref,
               w21_ref, b21_ref, w22_ref, b22_ref, mu_ref, lv_ref,
               agg_buf, u_buf, acc0, acc1):
    p = pl.program_id(0)
    i = pl.program_id(1)
    rows = pl.ds(i * _HBLK, _HBLK)

    @pl.when(p == 0)
    def _():
        a = jnp.concatenate([p_ref[0], p_ref[1]], axis=1)
        agg_buf[rows, :] = a
        st = jnp.concatenate(
            [jnp.sum(a, axis=0, keepdims=True),
             jnp.sum(a * a, axis=0, keepdims=True)], axis=0)

        @pl.when(i == 0)
        def _():
            acc0[...] = st

        @pl.when(i != 0)
        def _():
            acc0[...] = acc0[...] + st

    @pl.when(p == 1)
    def _():
        st = acc0[...]
        mean = st[0:1] * (1.0 / N)
        var = st[1:2] * (1.0 / N) - mean * mean
        inv = lax.rsqrt(var + _EPS)
        h0 = jax.nn.softplus(
            (agg_buf[rows, :] - mean) * inv * g0_ref[...] + b0_ref[...])
        u = (jnp.dot(h0, w1_ref[...], preferred_element_type=jnp.float32)
             + b1_ref[...])
        u_buf[rows, :] = u
        st1 = jnp.concatenate(
            [jnp.sum(u, axis=0, keepdims=True),
             jnp.sum(u * u, axis=0, keepdims=True)], axis=0)

        @pl.when(i == 0)
        def _():
            acc1[...] = st1

        @pl.when(i != 0)
        def _():
            acc1[...] = acc1[...] + st1

    @pl.when(p == 2)
    def _():
        st1 = acc1[...]
        mean = st1[0:1] * (1.0 / N)
        var = st1[1:2] * (1.0 / N) - mean * mean
        inv = lax.rsqrt(var + _EPS)
        h1 = jax.nn.softplus(
            (u_buf[rows, :] - mean) * inv * g1_ref[...] + b1n_ref[...])
        mu_ref[...] = (
            jnp.dot(h1, w21_ref[...], preferred_element_type=jnp.float32)
            + b21_ref[...])
        lv_ref[...] = (
            jnp.dot(h1, w22_ref[...], preferred_element_type=jnp.float32)
            + b22_ref[...])


def _head(parts, gamma0, beta0, W1, b1, gamma1, beta1, W21, b21, W22, b22):
    cst = lambda p, i: (0, 0)
    return pl.pallas_call(
        _head_body,
        out_shape=[
            jax.ShapeDtypeStruct((N, D_L), jnp.float32),
            jax.ShapeDtypeStruct((N, D_L), jnp.float32),
        ],
        grid=(3, _HNB),
        in_specs=[
            pl.BlockSpec((_NC, _HBLK, _DH2),
                         lambda p, i: (0, jnp.where(p == 0, i, 0), 0)),
            pl.BlockSpec((1, D_G), cst),
            pl.BlockSpec((1, D_G), cst),
            pl.BlockSpec((D_G, D_H), cst),
            pl.BlockSpec((1, D_H), cst),
            pl.BlockSpec((1, D_H), cst),
            pl.BlockSpec((1, D_H), cst),
            pl.BlockSpec((D_H, D_L), cst),
            pl.BlockSpec((1, D_L), cst),
            pl.BlockSpec((D_H, D_L), cst),
            pl.BlockSpec((1, D_L), cst),
        ],
        out_specs=[
            pl.BlockSpec((_HBLK, D_L), lambda p, i: (i, 0)),
            pl.BlockSpec((_HBLK, D_L), lambda p, i: (i, 0)),
        ],
        scratch_shapes=[
            pltpu.VMEM((N, D_G), jnp.float32),
            pltpu.VMEM((N, D_H), jnp.float32),
            pltpu.VMEM((2, D_G), jnp.float32),
            pltpu.VMEM((2, D_H), jnp.float32),
        ],
    )(parts, gamma0.reshape(1, -1), beta0.reshape(1, -1), W1,
      b1.reshape(1, -1), gamma1.reshape(1, -1), beta1.reshape(1, -1),
      W21, b21.reshape(1, -1), W22, b22.reshape(1, -1))


def kernel(x, edge_index, edge_weight, Wg, bg, gamma0, beta0, W1, b1,
           gamma1, beta1, W21, b21, W22, b22):
    hs = _mm_bias_split(x, Wg, bg)
    dst4 = edge_index[0].reshape(_NS, _NSEC, _SEC, _CH)
    src4 = edge_index[1].reshape(_NS, _NSEC, _SEC, _CH)
    w4 = edge_weight.reshape(_NS, _NSEC, _SEC, _CH)
    zeros_nd = jnp.zeros((N, _DH2), jnp.float32)
    parts = _sc_agg(hs, src4, dst4, w4, zeros_nd)
    mu, logvar = _head(parts, gamma0, beta0, W1, b1,
                       gamma1, beta1, W21, b21, W22, b22)
    return (mu, logvar)


# speedup vs baseline: 2.0037x; 1.9566x over previous
import functools

import jax
import jax.numpy as jnp
from jax import lax
from jax.experimental import pallas as pl
from jax.experimental.pallas import tpu as pltpu
from jax.experimental.pallas import tpu_sc as plsc

N = 10000
D_IN = 128
D_G = 128
D_H = 256
D_L = 64
E_TOT = 320000

_NC = 2
_NS = 16
_NW = _NC * _NS
_DH2 = D_G // _NC
_CH = 80
_RPT = E_TOT // _NS // _CH
_NSEC = 5
_SEC = _RPT // _NSEC
_EPS = 1e-5


def _mm_bias_body(x_ref, w_ref, b_ref, o_ref):
    u = (
        jnp.dot(x_ref[...], w_ref[...], preferred_element_type=jnp.float32)
        + b_ref[...]
    )
    o_ref[0] = u[:, :_DH2]
    o_ref[1] = u[:, _DH2:]


def _mm_bias_split(x, w, b):
    n, k = x.shape
    m = w.shape[1]
    nb = 10
    blk = n // nb
    return pl.pallas_call(
        _mm_bias_body,
        out_shape=jax.ShapeDtypeStruct((_NC, n, _DH2), jnp.float32),
        grid=(nb,),
        in_specs=[
            pl.BlockSpec((blk, k), lambda i: (i, 0)),
            pl.BlockSpec((k, m), lambda i: (0, 0)),
            pl.BlockSpec((1, m), lambda i: (0, 0)),
        ],
        out_specs=pl.BlockSpec((_NC, blk, _DH2), lambda i: (0, i, 0)),
    )(x, w, b.reshape(1, m))


def _sc_agg(hs, src4, dst4, w4, zeros_nd):
    mesh = plsc.VectorSubcoreMesh(core_axis_name="c", subcore_axis_name="s")

    @functools.partial(
        pl.kernel,
        out_type=jax.ShapeDtypeStruct((_NC, N, _DH2), jnp.float32),
        mesh=mesh,
        scratch_types=[
            pltpu.VMEM((_SEC, _CH), jnp.int32),
            pltpu.VMEM((_SEC, _CH), jnp.int32),
            pltpu.VMEM((_SEC, _CH), jnp.float32),
            pltpu.VMEM((_CH, _DH2), jnp.float32),
            pltpu.VMEM((_CH, _DH2), jnp.float32),
            pltpu.VMEM((_CH, _DH2), jnp.float32),
            pltpu.VMEM((_CH, _DH2), jnp.float32),
            pltpu.VMEM_SHARED((N, _DH2), jnp.float32),
            pltpu.SemaphoreType.DMA,
            pltpu.SemaphoreType.DMA,
            pltpu.SemaphoreType.DMA,
            pltpu.SemaphoreType.DMA,
        ],
        compiler_params=pltpu.CompilerParams(use_tc_tiling_on_sc=False),
    )
    def k(hs_hbm, src_hbm, dst_hbm, w_hbm, z_hbm, out_hbm,
          src_v, dst_v, w_v, gbuf0, gbuf1, sbuf0, sbuf1, acc,
          gsem0, gsem1, ssem0, ssem1):
        c = lax.axis_index("c")
        s = lax.axis_index("s")

        zb = s * 624
        pltpu.sync_copy(z_hbm.at[pl.ds(zb, 624)], acc.at[pl.ds(zb, 624)])

        @pl.when(s == _NS - 1)
        def _():
            pltpu.sync_copy(z_hbm.at[pl.ds(9984, 16)],
                            acc.at[pl.ds(9984, 16)])

        plsc.subcore_barrier()

        hhalf = hs_hbm.at[c]

        def start_gather(g, buf, sem):
            pltpu.async_copy(hhalf.at[src_v.at[g]], buf, sem)

        def wait_gather(buf, sem):
            pltpu.make_async_copy(hhalf.at[src_v.at[0]], buf, sem).wait()

        def multiply(g, gbuf, sbuf):
            def egroup(g16, _):
                w16 = w_v[g, pl.ds(g16 * 16, 16)]
                for l in range(16):
                    wv = jnp.full((16,), w16[l], dtype=jnp.float32)
                    r = g16 * 16 + l
                    for jj in range(_DH2 // 16):
                        sl = pl.ds(jj * 16, 16)
                        sbuf[r, sl] = gbuf[r, sl] * wv
                return 0
            lax.fori_loop(0, _CH // 16, egroup, 0)

        def start_scatter(g, buf, sem):
            pltpu.async_copy(buf, acc.at[dst_v.at[g]], sem, add=True)

        def wait_scatter(buf, sem):
            pltpu.make_async_copy(buf, acc.at[dst_v.at[0]], sem).wait()

        def section(sec, _):
            pltpu.sync_copy(src_hbm.at[s, sec], src_v)
            pltpu.sync_copy(dst_hbm.at[s, sec], dst_v)
            pltpu.sync_copy(w_hbm.at[s, sec], w_v)

            start_gather(0, gbuf0, gsem0)
            start_gather(1, gbuf1, gsem1)

            def pair(k2, _):
                g0 = 2 * k2

                wait_gather(gbuf0, gsem0)

                @pl.when(k2 > 0)
                def _():
                    wait_scatter(sbuf0, ssem0)

                multiply(g0, gbuf0, sbuf0)

                @pl.when(g0 + 2 < _SEC)
                def _():
                    start_gather(g0 + 2, gbuf0, gsem0)

                start_scatter(g0, sbuf0, ssem0)

                wait_gather(gbuf1, gsem1)

                @pl.when(k2 > 0)
                def _():
                    wait_scatter(sbuf1, ssem1)

                multiply(g0 + 1, gbuf1, sbuf1)

                @pl.when(g0 + 3 < _SEC)
                def _():
                    start_gather(g0 + 3, gbuf1, gsem1)

                start_scatter(g0 + 1, sbuf1, ssem1)
                return 0

            lax.fori_loop(0, _SEC // 2, pair, 0)
            wait_scatter(sbuf0, ssem0)
            wait_scatter(sbuf1, ssem1)
            return 0

        lax.fori_loop(0, _NSEC, section, 0)

        plsc.subcore_barrier()
        pltpu.sync_copy(acc.at[pl.ds(zb, 624)],
                        out_hbm.at[c].at[pl.ds(zb, 624)])

        @pl.when(s == _NS - 1)
        def _():
            pltpu.sync_copy(acc.at[pl.ds(9984, 16)],
                            out_hbm.at[c].at[pl.ds(9984, 16)])

    return k(hs, src4, dst4, w4, zeros_nd)


_HNB = 10
_HBLK = N // _HNB


def _head_body(p_ref, g0_ref, b0_ref, w1_ref, b1_ref, g1_ref, b1n_ref,
               w21_ref, b21_ref, w22_ref, b22_ref, mu_ref, lv_ref,
               agg_buf, u_buf, acc0, acc1):
    p = pl.program_id(0)
    i = pl.program_id(1)
    rows = pl.ds(i * _HBLK, _HBLK)

    @pl.when(p == 0)
    def _():
        a = jnp.concatenate([p_ref[0], p_ref[1]], axis=1)
        agg_buf[rows, :] = a
        st = jnp.concatenate(
            [jnp.sum(a, axis=0, keepdims=True),
             jnp.sum(a * a, axis=0, keepdims=True)], axis=0)

        @pl.when(i == 0)
        def _():
            acc0[...] = st

        @pl.when(i != 0)
        def _():
            acc0[...] = acc0[...] + st

    @pl.when(p == 1)
    def _():
        st = acc0[...]
        mean = st[0:1] * (1.0 / N)
        var = st[1:2] * (1.0 / N) - mean * mean
        inv = lax.rsqrt(var + _EPS)
        h0 = jax.nn.softplus(
            (agg_buf[rows, :] - mean) * inv * g0_ref[...] + b0_ref[...])
        u = (jnp.dot(h0, w1_ref[...], preferred_element_type=jnp.float32)
             + b1_ref[...])
        u_buf[rows, :] = u
        st1 = jnp.concatenate(
            [jnp.sum(u, axis=0, keepdims=True),
             jnp.sum(u * u, axis=0, keepdims=True)], axis=0)

        @pl.when(i == 0)
        def _():
            acc1[...] = st1

        @pl.when(i != 0)
        def _():
            acc1[...] = acc1[...] + st1

    @pl.when(p == 2)
    def _():
        st1 = acc1[...]
        mean = st1[0:1] * (1.0 / N)
        var = st1[1:2] * (1.0 / N) - mean * mean
        inv = lax.rsqrt(var + _EPS)
        h1 = jax.nn.softplus(
            (u_buf[rows, :] - mean) * inv * g1_ref[...] + b1n_ref[...])
        mu_ref[...] = (
            jnp.dot(h1, w21_ref[...], preferred_element_type=jnp.float32)
            + b21_ref[...])
        lv_ref[...] = (
            jnp.dot(h1, w22_ref[...], preferred_element_type=jnp.float32)
            + b22_ref[...])


def _head(parts, gamma0, beta0, W1, b1, gamma1, beta1, W21, b21, W22, b22):
    cst = lambda p, i: (0, 0)
    return pl.pallas_call(
        _head_body,
        out_shape=[
            jax.ShapeDtypeStruct((N, D_L), jnp.float32),
            jax.ShapeDtypeStruct((N, D_L), jnp.float32),
        ],
        grid=(3, _HNB),
        in_specs=[
            pl.BlockSpec((_NC, _HBLK, _DH2),
                         lambda p, i: (0, jnp.where(p == 0, i, 0), 0)),
            pl.BlockSpec((1, D_G), cst),
            pl.BlockSpec((1, D_G), cst),
            pl.BlockSpec((D_G, D_H), cst),
            pl.BlockSpec((1, D_H), cst),
            pl.BlockSpec((1, D_H), cst),
            pl.BlockSpec((1, D_H), cst),
            pl.BlockSpec((D_H, D_L), cst),
            pl.BlockSpec((1, D_L), cst),
            pl.BlockSpec((D_H, D_L), cst),
            pl.BlockSpec((1, D_L), cst),
        ],
        out_specs=[
            pl.BlockSpec((_HBLK, D_L), lambda p, i: (i, 0)),
            pl.BlockSpec((_HBLK, D_L), lambda p, i: (i, 0)),
        ],
        scratch_shapes=[
            pltpu.VMEM((N, D_G), jnp.float32),
            pltpu.VMEM((N, D_H), jnp.float32),
            pltpu.VMEM((2, D_G), jnp.float32),
            pltpu.VMEM((2, D_H), jnp.float32),
        ],
    )(parts, gamma0.reshape(1, -1), beta0.reshape(1, -1), W1,
      b1.reshape(1, -1), gamma1.reshape(1, -1), beta1.reshape(1, -1),
      W21, b21.reshape(1, -1), W22, b22.reshape(1, -1))


def kernel(x, edge_index, edge_weight, Wg, bg, gamma0, beta0, W1, b1,
           gamma1, beta1, W21, b21, W22, b22):
    hs = _mm_bias_split(x, Wg, bg)
    dst4 = edge_index[0].reshape(_NS, _NSEC, _SEC, _CH)
    src4 = edge_index[1].reshape(_NS, _NSEC, _SEC, _CH)
    w4 = edge_weight.reshape(_NS, _NSEC, _SEC, _CH)
    zeros_nd = jnp.zeros((N, _DH2), jnp.float32)
    parts = _sc_agg(hs, src4, dst4, w4, zeros_nd)
    mu, logvar = _head(parts, gamma0, beta0, W1, b1,
                       gamma1, beta1, W21, b21, W22, b22)
    return (mu, logvar)


# speedup vs baseline: 2.0865x; 1.0413x over previous
import functools

import jax
import jax.numpy as jnp
from jax import lax
from jax.experimental import pallas as pl
from jax.experimental.pallas import tpu as pltpu
from jax.experimental.pallas import tpu_sc as plsc

N = 10000
D_IN = 128
D_G = 128
D_H = 256
D_L = 64
E_TOT = 320000

_NC = 2
_NS = 16
_NW = _NC * _NS
_DH2 = D_G // _NC
_CH = 80
_RPT = E_TOT // _NS // _CH
_NSEC = 5
_SEC = _RPT // _NSEC
_EPS = 1e-5


def _mm_bias_body(x_ref, w_ref, b_ref, o_ref):
    u = (
        jnp.dot(x_ref[...].astype(jnp.bfloat16),
                w_ref[...].astype(jnp.bfloat16),
                preferred_element_type=jnp.float32)
        + b_ref[...]
    )
    o_ref[0] = u[:, :_DH2]
    o_ref[1] = u[:, _DH2:]


def _mm_bias_split(x, w, b):
    n, k = x.shape
    m = w.shape[1]
    nb = 10
    blk = n // nb
    return pl.pallas_call(
        _mm_bias_body,
        out_shape=jax.ShapeDtypeStruct((_NC, n, _DH2), jnp.float32),
        grid=(nb,),
        in_specs=[
            pl.BlockSpec((blk, k), lambda i: (i, 0)),
            pl.BlockSpec((k, m), lambda i: (0, 0)),
            pl.BlockSpec((1, m), lambda i: (0, 0)),
        ],
        out_specs=pl.BlockSpec((_NC, blk, _DH2), lambda i: (0, i, 0)),
    )(x, w, b.reshape(1, m))


def _sc_agg(hs, ei5, w4, zeros_nd):
    mesh = plsc.VectorSubcoreMesh(core_axis_name="c", subcore_axis_name="s")

    @functools.partial(
        pl.kernel,
        out_type=jax.ShapeDtypeStruct((_NC, N, _DH2), jnp.float32),
        mesh=mesh,
        scratch_types=[
            pltpu.VMEM((_SEC, _CH), jnp.int32),
            pltpu.VMEM((_SEC, _CH), jnp.int32),
            pltpu.VMEM((_SEC, _CH), jnp.float32),
            pltpu.VMEM((_CH, _DH2), jnp.float32),
            pltpu.VMEM((_CH, _DH2), jnp.float32),
            pltpu.VMEM((_CH, _DH2), jnp.float32),
            pltpu.VMEM((_CH, _DH2), jnp.float32),
            pltpu.VMEM_SHARED((N, _DH2), jnp.float32),
            pltpu.SemaphoreType.DMA,
            pltpu.SemaphoreType.DMA,
            pltpu.SemaphoreType.DMA,
            pltpu.SemaphoreType.DMA,
        ],
        compiler_params=pltpu.CompilerParams(use_tc_tiling_on_sc=False),
    )
    def k(hs_hbm, ei_hbm, w_hbm, z_hbm, out_hbm,
          src_v, dst_v, w_v, gbuf0, gbuf1, sbuf0, sbuf1, acc,
          gsem0, gsem1, ssem0, ssem1):
        c = lax.axis_index("c")
        s = lax.axis_index("s")

        zb = s * 624
        pltpu.sync_copy(z_hbm.at[pl.ds(zb, 624)], acc.at[pl.ds(zb, 624)])

        @pl.when(s == _NS - 1)
        def _():
            pltpu.sync_copy(z_hbm.at[pl.ds(9984, 16)],
                            acc.at[pl.ds(9984, 16)])

        plsc.subcore_barrier()

        hhalf = hs_hbm.at[c]

        def start_gather(g, buf, sem):
            pltpu.async_copy(hhalf.at[src_v.at[g]], buf, sem)

        def wait_gather(buf, sem):
            pltpu.make_async_copy(hhalf.at[src_v.at[0]], buf, sem).wait()

        def multiply(g, gbuf, sbuf):
            def egroup(g16, _):
                w16 = w_v[g, pl.ds(g16 * 16, 16)]
                for l in range(16):
                    wv = jnp.full((16,), w16[l], dtype=jnp.float32)
                    r = g16 * 16 + l
                    for jj in range(_DH2 // 16):
                        sl = pl.ds(jj * 16, 16)
                        sbuf[r, sl] = gbuf[r, sl] * wv
                return 0
            lax.fori_loop(0, _CH // 16, egroup, 0)

        def start_scatter(g, buf, sem):
            pltpu.async_copy(buf, acc.at[dst_v.at[g]], sem, add=True)

        def wait_scatter(buf, sem):
            pltpu.make_async_copy(buf, acc.at[dst_v.at[0]], sem).wait()

        def section(sec, _):
            pltpu.sync_copy(ei_hbm.at[1, s, sec], src_v)
            pltpu.sync_copy(ei_hbm.at[0, s, sec], dst_v)
            pltpu.sync_copy(w_hbm.at[s, sec], w_v)

            start_gather(0, gbuf0, gsem0)
            start_gather(1, gbuf1, gsem1)

            def pair(k2, _):
                g0 = 2 * k2

                wait_gather(gbuf0, gsem0)

                @pl.when(k2 > 0)
                def _():
                    wait_scatter(sbuf0, ssem0)

                multiply(g0, gbuf0, sbuf0)

                @pl.when(g0 + 2 < _SEC)
                def _():
                    start_gather(g0 + 2, gbuf0, gsem0)

                start_scatter(g0, sbuf0, ssem0)

                wait_gather(gbuf1, gsem1)

                @pl.when(k2 > 0)
                def _():
                    wait_scatter(sbuf1, ssem1)

                multiply(g0 + 1, gbuf1, sbuf1)

                @pl.when(g0 + 3 < _SEC)
                def _():
                    start_gather(g0 + 3, gbuf1, gsem1)

                start_scatter(g0 + 1, sbuf1, ssem1)
                return 0

            lax.fori_loop(0, _SEC // 2, pair, 0)
            wait_scatter(sbuf0, ssem0)
            wait_scatter(sbuf1, ssem1)
            return 0

        lax.fori_loop(0, _NSEC, section, 0)

        plsc.subcore_barrier()
        pltpu.sync_copy(acc.at[pl.ds(zb, 624)],
                        out_hbm.at[c].at[pl.ds(zb, 624)])

        @pl.when(s == _NS - 1)
        def _():
            pltpu.sync_copy(acc.at[pl.ds(9984, 16)],
                            out_hbm.at[c].at[pl.ds(9984, 16)])

    return k(hs, ei5, w4, zeros_nd)


_HNB = 10
_HBLK = N // _HNB


def _head_body(p_ref, g0_ref, b0_ref, w1_ref, b1_ref, g1_ref, b1n_ref,
               w21_ref, b21_ref, w22_ref, b22_ref, mu_ref, lv_ref,
               agg_buf, u_buf, acc0, acc1):
    p = pl.program_id(0)
    i = pl.program_id(1)
    rows = pl.ds(i * _HBLK, _HBLK)

    @pl.when(p == 0)
    def _():
        a = jnp.concatenate([p_ref[0], p_ref[1]], axis=1)
        agg_buf[rows, :] = a
        st = jnp.concatenate(
            [jnp.sum(a, axis=0, keepdims=True),
             jnp.sum(a * a, axis=0, keepdims=True)], axis=0)

        @pl.when(i == 0)
        def _():
            acc0[...] = st

        @pl.when(i != 0)
        def _():
            acc0[...] = acc0[...] + st

    @pl.when(p == 1)
    def _():
        st = acc0[...]
        mean = st[0:1] * (1.0 / N)
        var = st[1:2] * (1.0 / N) - mean * mean
        inv = lax.rsqrt(var + _EPS)
        h0 = jax.nn.softplus(
            (agg_buf[rows, :] - mean) * inv * g0_ref[...] + b0_ref[...])
        u = (jnp.dot(h0.astype(jnp.bfloat16),
                     w1_ref[...].astype(jnp.bfloat16),
                     preferred_element_type=jnp.float32)
             + b1_ref[...])
        u_buf[rows, :] = u
        st1 = jnp.concatenate(
            [jnp.sum(u, axis=0, keepdims=True),
             jnp.sum(u * u, axis=0, keepdims=True)], axis=0)

        @pl.when(i == 0)
        def _():
            acc1[...] = st1

        @pl.when(i != 0)
        def _():
            acc1[...] = acc1[...] + st1

    @pl.when(p == 2)
    def _():
        st1 = acc1[...]
        mean = st1[0:1] * (1.0 / N)
        var = st1[1:2] * (1.0 / N) - mean * mean
        inv = lax.rsqrt(var + _EPS)
        h1 = jax.nn.softplus(
            (u_buf[rows, :] - mean) * inv * g1_ref[...] + b1n_ref[...])
        h1b = h1.astype(jnp.bfloat16)
        mu_ref[...] = (
            jnp.dot(h1b, w21_ref[...].astype(jnp.bfloat16),
                    preferred_element_type=jnp.float32)
            + b21_ref[...])
        lv_ref[...] = (
            jnp.dot(h1b, w22_ref[...].astype(jnp.bfloat16),
                    preferred_element_type=jnp.float32)
            + b22_ref[...])


def _head(parts, gamma0, beta0, W1, b1, gamma1, beta1, W21, b21, W22, b22):
    cst = lambda p, i: (0, 0)
    return pl.pallas_call(
        _head_body,
        out_shape=[
            jax.ShapeDtypeStruct((N, D_L), jnp.float32),
            jax.ShapeDtypeStruct((N, D_L), jnp.float32),
        ],
        grid=(3, _HNB),
        in_specs=[
            pl.BlockSpec((_NC, _HBLK, _DH2),
                         lambda p, i: (0, jnp.where(p == 0, i, 0), 0)),
            pl.BlockSpec((1, D_G), cst),
            pl.BlockSpec((1, D_G), cst),
            pl.BlockSpec((D_G, D_H), cst),
            pl.BlockSpec((1, D_H), cst),
            pl.BlockSpec((1, D_H), cst),
            pl.BlockSpec((1, D_H), cst),
            pl.BlockSpec((D_H, D_L), cst),
            pl.BlockSpec((1, D_L), cst),
            pl.BlockSpec((D_H, D_L), cst),
            pl.BlockSpec((1, D_L), cst),
        ],
        out_specs=[
            pl.BlockSpec((_HBLK, D_L), lambda p, i: (i, 0)),
            pl.BlockSpec((_HBLK, D_L), lambda p, i: (i, 0)),
        ],
        scratch_shapes=[
            pltpu.VMEM((N, D_G), jnp.float32),
            pltpu.VMEM((N, D_H), jnp.float32),
            pltpu.VMEM((2, D_G), jnp.float32),
            pltpu.VMEM((2, D_H), jnp.float32),
        ],
    )(parts, gamma0.reshape(1, -1), beta0.reshape(1, -1), W1,
      b1.reshape(1, -1), gamma1.reshape(1, -1), beta1.reshape(1, -1),
      W21, b21.reshape(1, -1), W22, b22.reshape(1, -1))


def kernel(x, edge_index, edge_weight, Wg, bg, gamma0, beta0, W1, b1,
           gamma1, beta1, W21, b21, W22, b22):
    hs = _mm_bias_split(x, Wg, bg)
    ei5 = edge_index.reshape(2, _NS, _NSEC, _SEC, _CH)
    w4 = edge_weight.reshape(_NS, _NSEC, _SEC, _CH)
    zeros_nd = jnp.zeros((N, _DH2), jnp.float32)
    parts = _sc_agg(hs, ei5, w4, zeros_nd)
    mu, logvar = _head(parts, gamma0, beta0, W1, b1,
                       gamma1, beta1, W21, b21, W22, b22)
    return (mu, logvar)
